# Initial kernel scaffold; baseline (speedup 1.0000x reference)
import jax
import jax.numpy as jnp
from jax.experimental import pallas as pl


def kernel(tokens, focal_table, context_table):
    raise NotImplementedError("write your pallas kernel here")


# speedup vs baseline: 1.3694x; 1.3694x over previous
import functools

import jax
import jax.numpy as jnp
from jax import lax
from jax.experimental import pallas as pl
from jax.experimental.pallas import tpu as pltpu
from jax.experimental.pallas import tpu_sc as plsc

_NC = 2
_NS = 16
_LANES = 16


def kernel(tokens, focal_table, context_table):
    B, L = tokens.shape
    V, D = focal_table.shape
    N = B * L
    NW = _NC * _NS
    C = 512
    b_per_w = N // NW
    n_chunks = b_per_w // C
    assert b_per_w * NW == N and n_chunks * C == b_per_w

    idx = tokens.reshape(N).astype(jnp.int32)
    mesh = plsc.VectorSubcoreMesh(core_axis_name="c", subcore_axis_name="s")

    @functools.partial(
        pl.kernel,
        out_type=jax.ShapeDtypeStruct((N, D), jnp.float32),
        mesh=mesh,
        scratch_types=[
            pltpu.VMEM((C,), jnp.int32),
            pltpu.VMEM((C, D), jnp.float32),
            pltpu.VMEM((C, D), jnp.float32),
            pltpu.SemaphoreType.DMA,
            pltpu.SemaphoreType.DMA,
        ],
        compiler_params=pltpu.CompilerParams(use_tc_tiling_on_sc=False),
    )
    def sc_kernel(idx_hbm, focal_hbm, context_hbm, out_hbm,
                  idx_v, f_v, c_v, sem_f, sem_c):
        wid = lax.axis_index("s") * _NC + lax.axis_index("c")
        base = wid * b_per_w

        @pl.loop(0, n_chunks)
        def _(g):
            start = base + g * C
            pltpu.sync_copy(idx_hbm.at[pl.ds(start, C)], idx_v)
            cp_f = pltpu.async_copy(focal_hbm.at[idx_v], f_v, sem_f)
            cp_c = pltpu.async_copy(context_hbm.at[idx_v], c_v, sem_c)
            cp_f.wait()
            cp_c.wait()

            @pl.loop(0, C)
            def _(r):
                @pl.loop(0, D, step=_LANES)
                def _(j):
                    plsc.addupdate(f_v.at[r, pl.ds(j, _LANES)],
                                   c_v[r, pl.ds(j, _LANES)])

            pltpu.sync_copy(f_v, out_hbm.at[pl.ds(start, C)])

    out = sc_kernel(idx, focal_table, context_table)
    return out.reshape(B, L, D)


# speedup vs baseline: 1.4585x; 1.0651x over previous
import functools

import jax
import jax.numpy as jnp
from jax import lax
from jax.experimental import pallas as pl
from jax.experimental.pallas import tpu as pltpu
from jax.experimental.pallas import tpu_sc as plsc

_NC = 2
_NS = 16
_LANES = 16


def kernel(tokens, focal_table, context_table):
    B, L = tokens.shape
    V, D = focal_table.shape
    N = B * L
    NW = _NC * _NS
    C = 512
    b_per_w = N // NW
    n_chunks = b_per_w // C
    assert b_per_w * NW == N and n_chunks * C == b_per_w and n_chunks % 2 == 0

    idx = tokens.reshape(N).astype(jnp.int32)
    mesh = plsc.VectorSubcoreMesh(core_axis_name="c", subcore_axis_name="s")

    @functools.partial(
        pl.kernel,
        out_type=jax.ShapeDtypeStruct((N, D), jnp.float32),
        mesh=mesh,
        scratch_types=[
            pltpu.VMEM((b_per_w,), jnp.int32),
            pltpu.VMEM((C, D), jnp.float32),
            pltpu.VMEM((C, D), jnp.float32),
            pltpu.VMEM((C, D), jnp.float32),
            pltpu.VMEM((C, D), jnp.float32),
            pltpu.SemaphoreType.DMA,
            pltpu.SemaphoreType.DMA,
        ],
        compiler_params=pltpu.CompilerParams(use_tc_tiling_on_sc=False),
    )
    def sc_kernel(idx_hbm, focal_hbm, context_hbm, out_hbm,
                  idx_v, f0, c0, f1, c1, sem0, sem1):
        wid = lax.axis_index("s") * _NC + lax.axis_index("c")
        base = wid * b_per_w
        pltpu.sync_copy(idx_hbm.at[pl.ds(base, b_per_w)], idx_v)

        f_bufs, c_bufs, sems = (f0, f1), (c0, c1), (sem0, sem1)

        def issue(g, slot):
            sl = idx_v.at[pl.ds(g * C, C)]
            pltpu.async_copy(focal_hbm.at[sl], f_bufs[slot], sems[slot])
            pltpu.async_copy(context_hbm.at[sl], c_bufs[slot], sems[slot])

        def drain(g, slot):
            sl = idx_v.at[pl.ds(g * C, C)]
            pltpu.make_async_copy(focal_hbm.at[sl], f_bufs[slot],
                                  sems[slot]).wait()
            pltpu.make_async_copy(context_hbm.at[sl], c_bufs[slot],
                                  sems[slot]).wait()

        def process(g, slot):
            f_b, c_b = f_bufs[slot], c_bufs[slot]

            @plsc.parallel_loop(0, C, step=1, unroll=8)
            def _(r):
                plsc.addupdate(f_b.at[r, pl.ds(0, _LANES)],
                               c_b[r, pl.ds(0, _LANES)])
                plsc.addupdate(f_b.at[r, pl.ds(_LANES, _LANES)],
                               c_b[r, pl.ds(_LANES, _LANES)])

            pltpu.sync_copy(f_b, out_hbm.at[pl.ds(base + g * C, C)])

        issue(0, 0)
        issue(1, 1)

        @pl.loop(0, n_chunks, step=2)
        def _(g):
            drain(g, 0)
            process(g, 0)

            @pl.when(g + 2 < n_chunks)
            def _():
                issue(g + 2, 0)

            drain(g + 1, 1)
            process(g + 1, 1)

            @pl.when(g + 3 < n_chunks)
            def _():
                issue(g + 3, 1)

    out = sc_kernel(idx, focal_table, context_table)
    return out.reshape(B, L, D)
